# Initial kernel scaffold; baseline (speedup 1.0000x reference)
import jax
import jax.numpy as jnp
from jax.experimental import pallas as pl


def kernel(X, EI, batch, num_graphs, W1, b1, W2, b2, W3, b3, Wm1, bm1, Wm2, bm2):
    raise NotImplementedError("write your pallas kernel here")


# speedup vs baseline: 18.2674x; 18.2674x over previous
import functools

import jax
import jax.numpy as jnp
from jax import lax
from jax.experimental import pallas as pl
from jax.experimental.pallas import tpu as pltpu
from jax.experimental.pallas import tpu_sc as plsc

N = 10000
E = 320000
D = 128
HALF = 64
G = 64
KC = 125
ERWS = E // KC
NTEC = 16
NSC = 2
DW = 8
RB = 1000
NBLK = N // RB
SLAB = 624
SLAB_LAST = N - SLAB * (NTEC - 1)

_F32 = jnp.float32
_SC_PARAMS = pltpu.CompilerParams(use_tc_tiling_on_sc=False)


def _mesh():
    return plsc.VectorSubcoreMesh(core_axis_name="c", subcore_axis_name="s")


def _per_tec_slab(s, emit):
    base = pl.multiple_of(s * SLAB, 8)

    @pl.when(s < NTEC - 1)
    def _():
        emit(base, SLAB)

    @pl.when(s == NTEC - 1)
    def _():
        emit(SLAB * (NTEC - 1), SLAB_LAST)


def _sc_degree(rowr, z8, ones125):
    rpt = ERWS // (NSC * NTEC)

    @functools.partial(
        pl.kernel,
        mesh=_mesh(),
        out_type=jax.ShapeDtypeStruct((NSC, N, DW), _F32),
        compiler_params=_SC_PARAMS,
        scratch_types=[
            pltpu.VMEM((rpt, KC), jnp.int32),
            pltpu.VMEM((KC, DW), _F32),
            pltpu.VMEM_SHARED((N, DW), _F32),
            pltpu.SemaphoreType.DMA,
        ],
    )
    def k(rowr_h, z_h, ones_h, out_h, idx_v, ones_v, acc, sem):
        c = lax.axis_index("c")
        s = lax.axis_index("s")
        wid = c * NTEC + s
        pltpu.sync_copy(rowr_h.at[pl.ds(pl.multiple_of(wid * rpt, 8), rpt)],
                        idx_v)
        pltpu.sync_copy(ones_h, ones_v)
        _per_tec_slab(s, lambda o, n: pltpu.sync_copy(
            z_h.at[pl.ds(o, n)], acc.at[pl.ds(o, n)]))
        plsc.subcore_barrier()

        @pl.loop(0, rpt)
        def _(j):
            pltpu.sync_copy(ones_v, acc.at[idx_v.at[j]], add=True)

        plsc.subcore_barrier()
        _per_tec_slab(s, lambda o, n: pltpu.sync_copy(
            acc.at[pl.ds(o, n)], out_h.at[c, pl.ds(o, n)]))

    return k(rowr, z8, ones125)


def _sc_prop(h2a, h2b, colr, rowr):
    rpt = ERWS // NTEC

    @functools.partial(
        pl.kernel,
        mesh=_mesh(),
        out_type=(jax.ShapeDtypeStruct((N, HALF), _F32),
                  jax.ShapeDtypeStruct((N, HALF), _F32)),
        compiler_params=_SC_PARAMS,
        scratch_types=[
            pltpu.VMEM((rpt, KC), jnp.int32),
            pltpu.VMEM((rpt, KC), jnp.int32),
            pltpu.VMEM((KC, HALF), _F32),
            pltpu.VMEM((KC, HALF), _F32),
            pltpu.VMEM_SHARED((N, HALF), _F32),
            pltpu.SemaphoreType.DMA,
            pltpu.SemaphoreType.DMA,
        ],
    )
    def k(h2a_h, h2b_h, colr_h, rowr_h, oa_h, ob_h,
          col_v, row_v, msg0, msg1, acc, sem0, sem1):
        c = lax.axis_index("c")
        s = lax.axis_index("s")
        e0 = pl.multiple_of(s * rpt, 8)
        pltpu.sync_copy(colr_h.at[pl.ds(e0, rpt)], col_v)
        pltpu.sync_copy(rowr_h.at[pl.ds(e0, rpt)], row_v)
        bufs = (msg0, msg1)
        sems = (sem0, sem1)
        for cid in range(NSC):
            @pl.when(c == cid)
            def _(cid=cid):
                h2 = (h2a_h, h2b_h)[cid]
                out = (oa_h, ob_h)[cid]
                _per_tec_slab(s, lambda o, n: pltpu.sync_copy(
                    h2.at[pl.ds(o, n)], acc.at[pl.ds(o, n)]))
                plsc.subcore_barrier()

                pltpu.async_copy(h2.at[col_v.at[0]], msg0, sem0)

                @pl.loop(0, rpt, step=2)
                def _(g):
                    for b in range(2):
                        j = g + b
                        nb = (b + 1) % 2

                        @pl.when(j + 1 < rpt)
                        def _():
                            pltpu.async_copy(h2.at[col_v.at[j + 1]],
                                             bufs[nb], sems[nb])

                        pltpu.make_async_copy(h2.at[col_v.at[j]],
                                              bufs[b], sems[b]).wait()
                        pltpu.sync_copy(bufs[b], acc.at[row_v.at[j]],
                                        add=True)

                plsc.subcore_barrier()
                _per_tec_slab(s, lambda o, n: pltpu.sync_copy(
                    acc.at[pl.ds(o, n)], out.at[pl.ds(o, n)]))

    return k(h2a, h2b, colr, rowr)


def _dis_from_deg(deg_ref):
    deg = deg_ref[0, :, :1] + deg_ref[1, :, :1] + (1.0 + 1e-12)
    return lax.rsqrt(deg)


def _tc_stage(X, sa, sb, deg2, W, br, fl):
    def body(x_ref, sa_ref, sb_ref, deg_ref, w_ref, b_ref, f_ref,
             oa_ref, ob_ref):
        dis = _dis_from_deg(deg_ref)
        xr = jnp.concatenate([sa_ref[...], sb_ref[...]], axis=1) * dis
        xr = jnp.maximum(xr, 0.0)
        x = jnp.where(f_ref[0, 0] > 0.5, x_ref[...], xr)
        h = jnp.dot(x, w_ref[...], preferred_element_type=_F32) + b_ref[...]
        h2 = h * dis
        oa_ref[...] = h2[:, :HALF]
        ob_ref[...] = h2[:, HALF:]

    return pl.pallas_call(
        body,
        grid=(NBLK,),
        in_specs=[
            pl.BlockSpec((RB, D), lambda i: (i, 0)),
            pl.BlockSpec((RB, HALF), lambda i: (i, 0)),
            pl.BlockSpec((RB, HALF), lambda i: (i, 0)),
            pl.BlockSpec((NSC, RB, DW), lambda i: (0, i, 0)),
            pl.BlockSpec((D, D), lambda i: (0, 0)),
            pl.BlockSpec((1, D), lambda i: (0, 0)),
            pl.BlockSpec((1, 1), lambda i: (0, 0)),
        ],
        out_specs=[pl.BlockSpec((RB, HALF), lambda i: (i, 0)),
                   pl.BlockSpec((RB, HALF), lambda i: (i, 0))],
        out_shape=[jax.ShapeDtypeStruct((N, HALF), _F32)] * 2,
    )(X, sa, sb, deg2, W, br, fl)


def _tc_pool_mlp(sa, sb, deg2, brow, bcol, Wm1, bm1r, Wm2, bm2r, C):
    def body(sa_ref, sb_ref, deg_ref, brow_ref, bcol_ref,
             wm1_ref, bm1_ref, wm2_ref, bm2_ref, out_ref,
             sum_s, max_s, cnt_s):
        i = pl.program_id(0)

        @pl.when(i == 0)
        def _():
            sum_s[...] = jnp.zeros_like(sum_s)
            cnt_s[...] = jnp.zeros_like(cnt_s)
            max_s[...] = jnp.full_like(max_s, -jnp.inf)

        dis = _dis_from_deg(deg_ref)
        x = jnp.concatenate([sa_ref[...], sb_ref[...]], axis=1) * dis
        x = jnp.maximum(x, 0.0)

        brw = brow_ref[0]
        oneh = (lax.broadcasted_iota(jnp.int32, (G, RB), 0) == brw
                ).astype(_F32)
        sum_s[...] += jnp.dot(oneh, x, preferred_element_type=_F32)
        cnt_s[...] += jnp.sum(oneh, axis=1, keepdims=True)

        bcl = bcol_ref[...]

        def gbody(g, _):
            m = jnp.where(bcl == g, x, -jnp.inf)
            mg = jnp.max(m, axis=0, keepdims=True)
            max_s[pl.ds(g, 1), :] = jnp.maximum(max_s[pl.ds(g, 1), :], mg)
            return 0

        lax.fori_loop(0, G, gbody, 0)

        @pl.when(i == NBLK - 1)
        def _():
            mean = sum_s[...] / (cnt_s[...] + 1e-12)
            g64 = jnp.concatenate([mean, max_s[...]], axis=1)
            h = jnp.maximum(
                jnp.dot(g64, wm1_ref[...], preferred_element_type=_F32)
                + bm1_ref[...], 0.0)
            out_ref[...] = (jnp.dot(h, wm2_ref[...],
                                    preferred_element_type=_F32)
                            + bm2_ref[...])

    return pl.pallas_call(
        body,
        grid=(NBLK,),
        in_specs=[
            pl.BlockSpec((RB, HALF), lambda i: (i, 0)),
            pl.BlockSpec((RB, HALF), lambda i: (i, 0)),
            pl.BlockSpec((NSC, RB, DW), lambda i: (0, i, 0)),
            pl.BlockSpec((1, 1, RB), lambda i: (i, 0, 0)),
            pl.BlockSpec((RB, 1), lambda i: (i, 0)),
            pl.BlockSpec((2 * D, D), lambda i: (0, 0)),
            pl.BlockSpec((1, D), lambda i: (0, 0)),
            pl.BlockSpec((D, C), lambda i: (0, 0)),
            pl.BlockSpec((1, C), lambda i: (0, 0)),
        ],
        out_specs=pl.BlockSpec((G, C), lambda i: (0, 0)),
        out_shape=jax.ShapeDtypeStruct((G, C), _F32),
        scratch_shapes=[pltpu.VMEM((G, D), _F32),
                        pltpu.VMEM((G, D), _F32),
                        pltpu.VMEM((G, 1), _F32)],
    )(sa, sb, deg2, brow, bcol, Wm1, bm1r, Wm2, bm2r)


def kernel(X, EI, batch, num_graphs,
           W1, b1, W2, b2, W3, b3, Wm1, bm1, Wm2, bm2):
    C = Wm2.shape[1]
    row = EI[0]
    col = EI[1]
    rowr = row.reshape(ERWS, KC)
    colr = col.reshape(ERWS, KC)
    z8 = jnp.zeros((N, DW), _F32)
    z64 = jnp.zeros((N, HALF), _F32)
    ones125 = jnp.ones((KC, DW), _F32)

    deg2 = _sc_degree(rowr, z8, ones125)

    Wstack = jnp.stack([W1, W2, W3])
    bstack = jnp.stack([b1.reshape(1, -1), b2.reshape(1, -1),
                        b3.reshape(1, -1)])
    fstack = jnp.asarray([1.0, 0.0, 0.0], _F32).reshape(3, 1, 1)

    def step(carry, inp):
        sa, sb = carry
        W, br, fl = inp
        h2a, h2b = _tc_stage(X, sa, sb, deg2, W, br, fl)
        return _sc_prop(h2a, h2b, colr, rowr), None

    (sa, sb), _ = lax.scan(step, (z64, z64), (Wstack, bstack, fstack))

    return _tc_pool_mlp(sa, sb, deg2, batch.reshape(NBLK, 1, RB),
                        batch.reshape(N, 1), Wm1, bm1.reshape(1, -1),
                        Wm2, bm2.reshape(1, -1), C)


# speedup vs baseline: 20.2231x; 1.1071x over previous
import functools

import jax
import jax.numpy as jnp
from jax import lax
from jax.experimental import pallas as pl
from jax.experimental.pallas import tpu as pltpu
from jax.experimental.pallas import tpu_sc as plsc

N = 10000
E = 320000
D = 128
HALF = 64
G = 64
KC = 125
ERWS = E // KC
NTEC = 16
NSC = 2
DW = 8
RB = 1000
NBLK = N // RB
SLAB = 624
SLAB_LAST = N - SLAB * (NTEC - 1)

_F32 = jnp.float32
_SC_PARAMS = pltpu.CompilerParams(use_tc_tiling_on_sc=False)


def _mesh():
    return plsc.VectorSubcoreMesh(core_axis_name="c", subcore_axis_name="s")


def _per_tec_slab(s, emit):
    base = pl.multiple_of(s * SLAB, 8)

    @pl.when(s < NTEC - 1)
    def _():
        emit(base, SLAB)

    @pl.when(s == NTEC - 1)
    def _():
        emit(SLAB * (NTEC - 1), SLAB_LAST)


def _sc_degree(rowr, z8, ones125):
    rpt = ERWS // (NSC * NTEC)

    @functools.partial(
        pl.kernel,
        mesh=_mesh(),
        out_type=jax.ShapeDtypeStruct((NSC, N, DW), _F32),
        compiler_params=_SC_PARAMS,
        scratch_types=[
            pltpu.VMEM((rpt, KC), jnp.int32),
            pltpu.VMEM((KC, DW), _F32),
            pltpu.VMEM_SHARED((N, DW), _F32),
            pltpu.SemaphoreType.DMA,
        ],
    )
    def k(rowr_h, z_h, ones_h, out_h, idx_v, ones_v, acc, sem):
        c = lax.axis_index("c")
        s = lax.axis_index("s")
        wid = c * NTEC + s
        pltpu.sync_copy(rowr_h.at[pl.ds(pl.multiple_of(wid * rpt, 8), rpt)],
                        idx_v)
        pltpu.sync_copy(ones_h, ones_v)
        _per_tec_slab(s, lambda o, n: pltpu.sync_copy(
            z_h.at[pl.ds(o, n)], acc.at[pl.ds(o, n)]))
        plsc.subcore_barrier()

        @pl.loop(0, rpt)
        def _(j):
            pltpu.sync_copy(ones_v, acc.at[idx_v.at[j]], add=True)

        plsc.subcore_barrier()
        _per_tec_slab(s, lambda o, n: pltpu.sync_copy(
            acc.at[pl.ds(o, n)], out_h.at[c, pl.ds(o, n)]))

    return k(rowr, z8, ones125)


def _sc_prop(h2a, h2b, colr, rowr):
    rpt = ERWS // NTEC

    @functools.partial(
        pl.kernel,
        mesh=_mesh(),
        out_type=(jax.ShapeDtypeStruct((N, HALF), _F32),
                  jax.ShapeDtypeStruct((N, HALF), _F32)),
        compiler_params=_SC_PARAMS,
        scratch_types=[
            pltpu.VMEM((rpt, KC), jnp.int32),
            pltpu.VMEM((rpt, KC), jnp.int32),
            [pltpu.VMEM((KC, HALF), _F32)] * 4,
            [pltpu.SemaphoreType.DMA] * 4,
            [pltpu.SemaphoreType.DMA] * 4,
            pltpu.VMEM_SHARED((N, HALF), _F32),
        ],
    )
    def k(h2a_h, h2b_h, colr_h, rowr_h, oa_h, ob_h,
          col_v, row_v, bufs, gsems, ssems, acc):
        c = lax.axis_index("c")
        s = lax.axis_index("s")
        e0 = pl.multiple_of(s * rpt, 8)
        pltpu.sync_copy(colr_h.at[pl.ds(e0, rpt)], col_v)
        pltpu.sync_copy(rowr_h.at[pl.ds(e0, rpt)], row_v)
        for cid in range(NSC):
            @pl.when(c == cid)
            def _(cid=cid):
                h2 = (h2a_h, h2b_h)[cid]
                out = (oa_h, ob_h)[cid]
                _per_tec_slab(s, lambda o, n: pltpu.sync_copy(
                    h2.at[pl.ds(o, n)], acc.at[pl.ds(o, n)]))
                plsc.subcore_barrier()

                pltpu.async_copy(h2.at[col_v.at[0]], bufs[0], gsems[0])
                pltpu.async_copy(h2.at[col_v.at[1]], bufs[1], gsems[1])

                @pl.loop(0, rpt, step=4)
                def _(base):
                    for b in range(4):
                        j = base + b
                        g = j + 2
                        bg = (b + 2) % 4

                        @pl.when(g < rpt)
                        def _():
                            @pl.when(g >= 4)
                            def _():
                                pltpu.make_async_copy(
                                    bufs[bg], acc.at[row_v.at[g - 4]],
                                    ssems[bg]).wait()

                            pltpu.async_copy(h2.at[col_v.at[g]],
                                             bufs[bg], gsems[bg])

                        pltpu.make_async_copy(h2.at[col_v.at[j]],
                                              bufs[b], gsems[b]).wait()
                        pltpu.async_copy(bufs[b], acc.at[row_v.at[j]],
                                         ssems[b], add=True)

                for b in range(4):
                    pltpu.make_async_copy(
                        bufs[b], acc.at[row_v.at[rpt - 4 + b]],
                        ssems[b]).wait()

                plsc.subcore_barrier()
                _per_tec_slab(s, lambda o, n: pltpu.sync_copy(
                    acc.at[pl.ds(o, n)], out.at[pl.ds(o, n)]))

    return k(h2a, h2b, colr, rowr)


def _dis_from_deg(deg_ref):
    deg = deg_ref[0, :, :1] + deg_ref[1, :, :1] + (1.0 + 1e-12)
    return lax.rsqrt(deg)


def _tc_stage(X, sa, sb, deg2, W, br, fl):
    def body(x_ref, sa_ref, sb_ref, deg_ref, w_ref, b_ref, f_ref,
             oa_ref, ob_ref):
        dis = _dis_from_deg(deg_ref)
        xr = jnp.concatenate([sa_ref[...], sb_ref[...]], axis=1) * dis
        xr = jnp.maximum(xr, 0.0)
        x = jnp.where(f_ref[0, 0] > 0.5, x_ref[...], xr)
        h = jnp.dot(x, w_ref[...], preferred_element_type=_F32) + b_ref[...]
        h2 = h * dis
        oa_ref[...] = h2[:, :HALF]
        ob_ref[...] = h2[:, HALF:]

    return pl.pallas_call(
        body,
        grid=(NBLK,),
        in_specs=[
            pl.BlockSpec((RB, D), lambda i: (i, 0)),
            pl.BlockSpec((RB, HALF), lambda i: (i, 0)),
            pl.BlockSpec((RB, HALF), lambda i: (i, 0)),
            pl.BlockSpec((NSC, RB, DW), lambda i: (0, i, 0)),
            pl.BlockSpec((D, D), lambda i: (0, 0)),
            pl.BlockSpec((1, D), lambda i: (0, 0)),
            pl.BlockSpec((1, 1), lambda i: (0, 0)),
        ],
        out_specs=[pl.BlockSpec((RB, HALF), lambda i: (i, 0)),
                   pl.BlockSpec((RB, HALF), lambda i: (i, 0))],
        out_shape=[jax.ShapeDtypeStruct((N, HALF), _F32)] * 2,
    )(X, sa, sb, deg2, W, br, fl)


def _tc_pool_mlp(sa, sb, deg2, brow, bcol, Wm1, bm1r, Wm2, bm2r, C):
    def body(sa_ref, sb_ref, deg_ref, brow_ref, bcol_ref,
             wm1_ref, bm1_ref, wm2_ref, bm2_ref, out_ref,
             sum_s, max_s, cnt_s):
        i = pl.program_id(0)

        @pl.when(i == 0)
        def _():
            sum_s[...] = jnp.zeros_like(sum_s)
            cnt_s[...] = jnp.zeros_like(cnt_s)
            max_s[...] = jnp.full_like(max_s, -jnp.inf)

        dis = _dis_from_deg(deg_ref)
        x = jnp.concatenate([sa_ref[...], sb_ref[...]], axis=1) * dis
        x = jnp.maximum(x, 0.0)

        brw = brow_ref[0]
        oneh = (lax.broadcasted_iota(jnp.int32, (G, RB), 0) == brw
                ).astype(_F32)
        sum_s[...] += jnp.dot(oneh, x, preferred_element_type=_F32)
        cnt_s[...] += jnp.sum(oneh, axis=1, keepdims=True)

        bcl = bcol_ref[...]

        def gbody(g, _):
            m = jnp.where(bcl == g, x, -jnp.inf)
            mg = jnp.max(m, axis=0, keepdims=True)
            max_s[pl.ds(g, 1), :] = jnp.maximum(max_s[pl.ds(g, 1), :], mg)
            return 0

        lax.fori_loop(0, G, gbody, 0)

        @pl.when(i == NBLK - 1)
        def _():
            mean = sum_s[...] / (cnt_s[...] + 1e-12)
            g64 = jnp.concatenate([mean, max_s[...]], axis=1)
            h = jnp.maximum(
                jnp.dot(g64, wm1_ref[...], preferred_element_type=_F32)
                + bm1_ref[...], 0.0)
            out_ref[...] = (jnp.dot(h, wm2_ref[...],
                                    preferred_element_type=_F32)
                            + bm2_ref[...])

    return pl.pallas_call(
        body,
        grid=(NBLK,),
        in_specs=[
            pl.BlockSpec((RB, HALF), lambda i: (i, 0)),
            pl.BlockSpec((RB, HALF), lambda i: (i, 0)),
            pl.BlockSpec((NSC, RB, DW), lambda i: (0, i, 0)),
            pl.BlockSpec((1, 1, RB), lambda i: (i, 0, 0)),
            pl.BlockSpec((RB, 1), lambda i: (i, 0)),
            pl.BlockSpec((2 * D, D), lambda i: (0, 0)),
            pl.BlockSpec((1, D), lambda i: (0, 0)),
            pl.BlockSpec((D, C), lambda i: (0, 0)),
            pl.BlockSpec((1, C), lambda i: (0, 0)),
        ],
        out_specs=pl.BlockSpec((G, C), lambda i: (0, 0)),
        out_shape=jax.ShapeDtypeStruct((G, C), _F32),
        scratch_shapes=[pltpu.VMEM((G, D), _F32),
                        pltpu.VMEM((G, D), _F32),
                        pltpu.VMEM((G, 1), _F32)],
    )(sa, sb, deg2, brow, bcol, Wm1, bm1r, Wm2, bm2r)


def kernel(X, EI, batch, num_graphs,
           W1, b1, W2, b2, W3, b3, Wm1, bm1, Wm2, bm2):
    C = Wm2.shape[1]
    row = EI[0]
    col = EI[1]
    rowr = row.reshape(ERWS, KC)
    colr = col.reshape(ERWS, KC)
    z8 = jnp.zeros((N, DW), _F32)
    z64 = jnp.zeros((N, HALF), _F32)
    ones125 = jnp.ones((KC, DW), _F32)

    deg2 = _sc_degree(rowr, z8, ones125)

    Wstack = jnp.stack([W1, W2, W3])
    bstack = jnp.stack([b1.reshape(1, -1), b2.reshape(1, -1),
                        b3.reshape(1, -1)])
    fstack = jnp.asarray([1.0, 0.0, 0.0], _F32).reshape(3, 1, 1)

    def step(carry, inp):
        sa, sb = carry
        W, br, fl = inp
        h2a, h2b = _tc_stage(X, sa, sb, deg2, W, br, fl)
        return _sc_prop(h2a, h2b, colr, rowr), None

    (sa, sb), _ = lax.scan(step, (z64, z64), (Wstack, bstack, fstack))

    return _tc_pool_mlp(sa, sb, deg2, batch.reshape(NBLK, 1, RB),
                        batch.reshape(N, 1), Wm1, bm1.reshape(1, -1),
                        Wm2, bm2.reshape(1, -1), C)


# speedup vs baseline: 25.4174x; 1.2568x over previous
import functools

import jax
import jax.numpy as jnp
from jax import lax
from jax.experimental import pallas as pl
from jax.experimental.pallas import tpu as pltpu
from jax.experimental.pallas import tpu_sc as plsc

N = 10000
E = 320000
D = 128
HALF = 64
G = 64
KC = 125
ERWS = E // KC
NTEC = 16
NSC = 2
DW = 8
RB = 1000
NBLK = N // RB
SLAB = 624
SLAB_LAST = N - SLAB * (NTEC - 1)

_F32 = jnp.float32
_SC_PARAMS = pltpu.CompilerParams(use_tc_tiling_on_sc=False)


def _mesh():
    return plsc.VectorSubcoreMesh(core_axis_name="c", subcore_axis_name="s")


def _per_tec_slab(s, emit):
    base = pl.multiple_of(s * SLAB, 8)

    @pl.when(s < NTEC - 1)
    def _():
        emit(base, SLAB)

    @pl.when(s == NTEC - 1)
    def _():
        emit(SLAB * (NTEC - 1), SLAB_LAST)


def _sc_degree(rowr, z8, ones125):
    rpt = ERWS // (NSC * NTEC)

    @functools.partial(
        pl.kernel,
        mesh=_mesh(),
        out_type=jax.ShapeDtypeStruct((NSC, N, DW), _F32),
        compiler_params=_SC_PARAMS,
        scratch_types=[
            pltpu.VMEM((rpt, KC), jnp.int32),
            pltpu.VMEM((KC, DW), _F32),
            pltpu.VMEM_SHARED((N, DW), _F32),
            pltpu.SemaphoreType.DMA,
        ],
    )
    def k(rowr_h, z_h, ones_h, out_h, idx_v, ones_v, acc, sem):
        c = lax.axis_index("c")
        s = lax.axis_index("s")
        wid = c * NTEC + s
        pltpu.sync_copy(rowr_h.at[pl.ds(pl.multiple_of(wid * rpt, 8), rpt)],
                        idx_v)
        pltpu.sync_copy(ones_h, ones_v)
        _per_tec_slab(s, lambda o, n: pltpu.sync_copy(
            z_h.at[pl.ds(o, n)], acc.at[pl.ds(o, n)]))
        plsc.subcore_barrier()

        @pl.loop(0, rpt)
        def _(j):
            pltpu.sync_copy(ones_v, acc.at[idx_v.at[j]], add=True)

        plsc.subcore_barrier()
        _per_tec_slab(s, lambda o, n: pltpu.sync_copy(
            acc.at[pl.ds(o, n)], out_h.at[c, pl.ds(o, n)]))

    return k(rowr, z8, ones125)


def _sc_prop(h2a, h2b, colr, rowr):
    rpt = ERWS // NTEC

    @functools.partial(
        pl.kernel,
        mesh=_mesh(),
        out_type=(jax.ShapeDtypeStruct((N, HALF), _F32),
                  jax.ShapeDtypeStruct((N, HALF), _F32)),
        compiler_params=_SC_PARAMS,
        scratch_types=[
            pltpu.VMEM((rpt, KC), jnp.int32),
            pltpu.VMEM((rpt, KC), jnp.int32),
            [pltpu.VMEM((KC, HALF), _F32)] * 4,
            [pltpu.SemaphoreType.DMA] * 4,
            [pltpu.SemaphoreType.DMA] * 4,
            pltpu.VMEM_SHARED((N, HALF), _F32),
        ],
    )
    def k(h2a_h, h2b_h, colr_h, rowr_h, oa_h, ob_h,
          col_v, row_v, bufs, gsems, ssems, acc):
        c = lax.axis_index("c")
        s = lax.axis_index("s")
        e0 = pl.multiple_of(s * rpt, 8)
        pltpu.sync_copy(colr_h.at[pl.ds(e0, rpt)], col_v)
        pltpu.sync_copy(rowr_h.at[pl.ds(e0, rpt)], row_v)
        for cid in range(NSC):
            @pl.when(c == cid)
            def _(cid=cid):
                h2 = (h2a_h, h2b_h)[cid]
                out = (oa_h, ob_h)[cid]
                _per_tec_slab(s, lambda o, n: pltpu.sync_copy(
                    h2.at[pl.ds(o, n)], acc.at[pl.ds(o, n)]))
                plsc.subcore_barrier()

                pltpu.async_copy(h2.at[col_v.at[0]], bufs[0], gsems[0])
                pltpu.async_copy(h2.at[col_v.at[1]], bufs[1], gsems[1])

                @pl.loop(0, rpt, step=4)
                def _(base):
                    for b in range(4):
                        j = base + b
                        g = j + 2
                        bg = (b + 2) % 4

                        @pl.when(g < rpt)
                        def _():
                            @pl.when(g >= 4)
                            def _():
                                pltpu.make_async_copy(
                                    bufs[bg], acc.at[row_v.at[g - 4]],
                                    ssems[bg]).wait()

                            pltpu.async_copy(h2.at[col_v.at[g]],
                                             bufs[bg], gsems[bg])

                        pltpu.make_async_copy(h2.at[col_v.at[j]],
                                              bufs[b], gsems[b]).wait()
                        pltpu.async_copy(bufs[b], acc.at[row_v.at[j]],
                                         ssems[b], add=True)

                for b in range(4):
                    pltpu.make_async_copy(
                        bufs[b], acc.at[row_v.at[rpt - 4 + b]],
                        ssems[b]).wait()

                plsc.subcore_barrier()
                _per_tec_slab(s, lambda o, n: pltpu.sync_copy(
                    acc.at[pl.ds(o, n)], out.at[pl.ds(o, n)]))

    return k(h2a, h2b, colr, rowr)


def _dis_from_deg(deg_ref):
    deg = deg_ref[0, :, :1] + deg_ref[1, :, :1] + (1.0 + 1e-12)
    return lax.rsqrt(deg)


def _tc_stage(X, sa, sb, deg2, W, br, fl):
    RBS = 2000
    NBS = N // RBS

    def body(x_ref, sa_ref, sb_ref, deg_ref, w_ref, b_ref, f_ref,
             oa_ref, ob_ref):
        dis = _dis_from_deg(deg_ref)
        xr = jnp.concatenate([sa_ref[...], sb_ref[...]], axis=1) * dis
        xr = jnp.maximum(xr, 0.0)
        x = jnp.where(f_ref[0, 0] > 0.5, x_ref[...], xr)
        h = jnp.dot(x, w_ref[...], preferred_element_type=_F32) + b_ref[...]
        h2 = h * dis
        oa_ref[...] = h2[:, :HALF]
        ob_ref[...] = h2[:, HALF:]

    return pl.pallas_call(
        body,
        grid=(NBS,),
        in_specs=[
            pl.BlockSpec((RBS, D), lambda i: (i, 0)),
            pl.BlockSpec((RBS, HALF), lambda i: (i, 0)),
            pl.BlockSpec((RBS, HALF), lambda i: (i, 0)),
            pl.BlockSpec((NSC, RBS, DW), lambda i: (0, i, 0)),
            pl.BlockSpec((D, D), lambda i: (0, 0)),
            pl.BlockSpec((1, D), lambda i: (0, 0)),
            pl.BlockSpec((1, 1), lambda i: (0, 0)),
        ],
        out_specs=[pl.BlockSpec((RBS, HALF), lambda i: (i, 0)),
                   pl.BlockSpec((RBS, HALF), lambda i: (i, 0))],
        out_shape=[jax.ShapeDtypeStruct((N, HALF), _F32)] * 2,
    )(X, sa, sb, deg2, W, br, fl)


def _tc_pool_mlp(sa, sb, deg2, brow, bcol, Wm1, bm1r, Wm2, bm2r, C):
    def body(sa_ref, sb_ref, deg_ref, brow_ref, bcol_ref,
             wm1_ref, bm1_ref, wm2_ref, bm2_ref, out_ref,
             sum_s, max_s, cnt_s):
        i = pl.program_id(0)

        @pl.when(i == 0)
        def _():
            sum_s[...] = jnp.zeros_like(sum_s)
            cnt_s[...] = jnp.zeros_like(cnt_s)
            max_s[...] = jnp.full_like(max_s, -jnp.inf)

        dis = _dis_from_deg(deg_ref)
        x = jnp.concatenate([sa_ref[...], sb_ref[...]], axis=1) * dis
        x = jnp.maximum(x, 0.0)

        brw = brow_ref[0]
        oneh = (lax.broadcasted_iota(jnp.int32, (G, RB), 0) == brw
                ).astype(_F32)
        sum_s[...] += jnp.dot(oneh, x, preferred_element_type=_F32)
        cnt_s[...] += jnp.sum(oneh, axis=1, keepdims=True)

        bcl = bcol_ref[...]

        def gbody(g, _):
            m = jnp.where(bcl == g, x, -jnp.inf)
            mg = jnp.max(m, axis=0, keepdims=True)
            max_s[pl.ds(g, 1), :] = jnp.maximum(max_s[pl.ds(g, 1), :], mg)
            return 0

        lax.fori_loop(bcol_ref[0, 0], bcol_ref[RB - 1, 0] + 1, gbody, 0,
                      unroll=False)

        @pl.when(i == NBLK - 1)
        def _():
            mean = sum_s[...] / (cnt_s[...] + 1e-12)
            g64 = jnp.concatenate([mean, max_s[...]], axis=1)
            h = jnp.maximum(
                jnp.dot(g64, wm1_ref[...], preferred_element_type=_F32)
                + bm1_ref[...], 0.0)
            out_ref[...] = (jnp.dot(h, wm2_ref[...],
                                    preferred_element_type=_F32)
                            + bm2_ref[...])

    return pl.pallas_call(
        body,
        grid=(NBLK,),
        in_specs=[
            pl.BlockSpec((RB, HALF), lambda i: (i, 0)),
            pl.BlockSpec((RB, HALF), lambda i: (i, 0)),
            pl.BlockSpec((NSC, RB, DW), lambda i: (0, i, 0)),
            pl.BlockSpec((1, 1, RB), lambda i: (i, 0, 0)),
            pl.BlockSpec((RB, 1), lambda i: (i, 0)),
            pl.BlockSpec((2 * D, D), lambda i: (0, 0)),
            pl.BlockSpec((1, D), lambda i: (0, 0)),
            pl.BlockSpec((D, C), lambda i: (0, 0)),
            pl.BlockSpec((1, C), lambda i: (0, 0)),
        ],
        out_specs=pl.BlockSpec((G, C), lambda i: (0, 0)),
        out_shape=jax.ShapeDtypeStruct((G, C), _F32),
        scratch_shapes=[pltpu.VMEM((G, D), _F32),
                        pltpu.VMEM((G, D), _F32),
                        pltpu.VMEM((G, 1), _F32)],
    )(sa, sb, deg2, brow, bcol, Wm1, bm1r, Wm2, bm2r)


def kernel(X, EI, batch, num_graphs,
           W1, b1, W2, b2, W3, b3, Wm1, bm1, Wm2, bm2):
    C = Wm2.shape[1]
    row = EI[0]
    col = EI[1]
    rowr = row.reshape(ERWS, KC)
    colr = col.reshape(ERWS, KC)
    z8 = jnp.zeros((N, DW), _F32)
    z64 = jnp.zeros((N, HALF), _F32)
    ones125 = jnp.ones((KC, DW), _F32)

    deg2 = _sc_degree(rowr, z8, ones125)

    Wstack = jnp.stack([W1, W2, W3])
    bstack = jnp.stack([b1.reshape(1, -1), b2.reshape(1, -1),
                        b3.reshape(1, -1)])
    fstack = jnp.asarray([1.0, 0.0, 0.0], _F32).reshape(3, 1, 1)

    def step(carry, inp):
        sa, sb = carry
        W, br, fl = inp
        h2a, h2b = _tc_stage(X, sa, sb, deg2, W, br, fl)
        return _sc_prop(h2a, h2b, colr, rowr), None

    (sa, sb), _ = lax.scan(step, (z64, z64), (Wstack, bstack, fstack))

    return _tc_pool_mlp(sa, sb, deg2, batch.reshape(NBLK, 1, RB),
                        batch.reshape(N, 1), Wm1, bm1.reshape(1, -1),
                        Wm2, bm2.reshape(1, -1), C)


# speedup vs baseline: 26.3750x; 1.0377x over previous
import functools

import jax
import jax.numpy as jnp
from jax import lax
from jax.experimental import pallas as pl
from jax.experimental.pallas import tpu as pltpu
from jax.experimental.pallas import tpu_sc as plsc

N = 10000
E = 320000
D = 128
HALF = 64
G = 64
KC = 125
ERWS = E // KC
NTEC = 16
NSC = 2
DW = 8
RB = 1000
NBLK = N // RB
SLAB = 624
SLAB_LAST = N - SLAB * (NTEC - 1)

_F32 = jnp.float32
_SC_PARAMS = pltpu.CompilerParams(use_tc_tiling_on_sc=False)


def _mesh():
    return plsc.VectorSubcoreMesh(core_axis_name="c", subcore_axis_name="s")


def _per_tec_slab(s, emit):
    base = pl.multiple_of(s * SLAB, 8)

    @pl.when(s < NTEC - 1)
    def _():
        emit(base, SLAB)

    @pl.when(s == NTEC - 1)
    def _():
        emit(SLAB * (NTEC - 1), SLAB_LAST)


def _sc_degree(rowr, z8, ones125):
    rpt = ERWS // (NSC * NTEC)

    @functools.partial(
        pl.kernel,
        mesh=_mesh(),
        out_type=jax.ShapeDtypeStruct((NSC, N, DW), _F32),
        compiler_params=_SC_PARAMS,
        scratch_types=[
            pltpu.VMEM((rpt, KC), jnp.int32),
            pltpu.VMEM((KC, DW), _F32),
            pltpu.VMEM_SHARED((N, DW), _F32),
            pltpu.SemaphoreType.DMA,
        ],
    )
    def k(rowr_h, z_h, ones_h, out_h, idx_v, ones_v, acc, sem):
        c = lax.axis_index("c")
        s = lax.axis_index("s")
        wid = c * NTEC + s
        pltpu.sync_copy(rowr_h.at[pl.ds(pl.multiple_of(wid * rpt, 8), rpt)],
                        idx_v)
        pltpu.sync_copy(ones_h, ones_v)
        _per_tec_slab(s, lambda o, n: pltpu.sync_copy(
            z_h.at[pl.ds(o, n)], acc.at[pl.ds(o, n)]))
        plsc.subcore_barrier()

        @pl.loop(0, rpt)
        def _(j):
            pltpu.sync_copy(ones_v, acc.at[idx_v.at[j]], add=True)

        plsc.subcore_barrier()
        _per_tec_slab(s, lambda o, n: pltpu.sync_copy(
            acc.at[pl.ds(o, n)], out_h.at[c, pl.ds(o, n)]))

    return k(rowr, z8, ones125)


def _sc_prop(h2a, h2b, colr, rowr):
    rpt = ERWS // NTEC

    @functools.partial(
        pl.kernel,
        mesh=_mesh(),
        out_type=(jax.ShapeDtypeStruct((N, HALF), _F32),
                  jax.ShapeDtypeStruct((N, HALF), _F32)),
        compiler_params=_SC_PARAMS,
        scratch_types=[
            pltpu.VMEM((rpt, KC), jnp.int32),
            pltpu.VMEM((rpt, KC), jnp.int32),
            [pltpu.VMEM((KC, HALF), _F32)] * 4,
            [pltpu.SemaphoreType.DMA] * 4,
            [pltpu.SemaphoreType.DMA] * 4,
            pltpu.VMEM_SHARED((N, HALF), _F32),
        ],
    )
    def k(h2a_h, h2b_h, colr_h, rowr_h, oa_h, ob_h,
          col_v, row_v, bufs, gsems, ssems, acc):
        c = lax.axis_index("c")
        s = lax.axis_index("s")
        e0 = pl.multiple_of(s * rpt, 8)
        pltpu.sync_copy(colr_h.at[pl.ds(e0, rpt)], col_v)
        pltpu.sync_copy(rowr_h.at[pl.ds(e0, rpt)], row_v)
        for cid in range(NSC):
            @pl.when(c == cid)
            def _(cid=cid):
                h2 = (h2a_h, h2b_h)[cid]
                out = (oa_h, ob_h)[cid]
                _per_tec_slab(s, lambda o, n: pltpu.sync_copy(
                    h2.at[pl.ds(o, n)], acc.at[pl.ds(o, n)]))
                plsc.subcore_barrier()

                pltpu.async_copy(h2.at[col_v.at[0]], bufs[0], gsems[0])
                pltpu.async_copy(h2.at[col_v.at[1]], bufs[1], gsems[1])

                @pl.loop(0, rpt, step=4)
                def _(base):
                    for b in range(4):
                        j = base + b
                        g = j + 2
                        bg = (b + 2) % 4

                        @pl.when(g < rpt)
                        def _():
                            @pl.when(g >= 4)
                            def _():
                                pltpu.make_async_copy(
                                    bufs[bg], acc.at[row_v.at[g - 4]],
                                    ssems[bg]).wait()

                            pltpu.async_copy(h2.at[col_v.at[g]],
                                             bufs[bg], gsems[bg])

                        pltpu.make_async_copy(h2.at[col_v.at[j]],
                                              bufs[b], gsems[b]).wait()
                        pltpu.async_copy(bufs[b], acc.at[row_v.at[j]],
                                         ssems[b], add=True)

                for b in range(4):
                    pltpu.make_async_copy(
                        bufs[b], acc.at[row_v.at[rpt - 4 + b]],
                        ssems[b]).wait()

                plsc.subcore_barrier()
                _per_tec_slab(s, lambda o, n: pltpu.sync_copy(
                    acc.at[pl.ds(o, n)], out.at[pl.ds(o, n)]))

    return k(h2a, h2b, colr, rowr)


def _dis_from_deg(deg_ref):
    deg = deg_ref[0, :, :1] + deg_ref[1, :, :1] + (1.0 + 1e-12)
    return lax.rsqrt(deg)


def _tc_stage(X, sa, sb, deg2, W, br, fl):
    RBS = 2000
    NBS = N // RBS

    def body(x_ref, sa_ref, sb_ref, deg_ref, w_ref, b_ref, f_ref,
             oa_ref, ob_ref):
        dis = _dis_from_deg(deg_ref)
        xr = jnp.concatenate([sa_ref[...], sb_ref[...]], axis=1) * dis
        xr = jnp.maximum(xr, 0.0)
        x = jnp.where(f_ref[0, 0] > 0.5, x_ref[...], xr)
        h = jnp.dot(x, w_ref[...], preferred_element_type=_F32) + b_ref[...]
        h2 = h * dis
        oa_ref[...] = h2[:, :HALF]
        ob_ref[...] = h2[:, HALF:]

    return pl.pallas_call(
        body,
        grid=(NBS,),
        in_specs=[
            pl.BlockSpec((RBS, D), lambda i: (i, 0)),
            pl.BlockSpec((RBS, HALF), lambda i: (i, 0)),
            pl.BlockSpec((RBS, HALF), lambda i: (i, 0)),
            pl.BlockSpec((NSC, RBS, DW), lambda i: (0, i, 0)),
            pl.BlockSpec((D, D), lambda i: (0, 0)),
            pl.BlockSpec((1, D), lambda i: (0, 0)),
            pl.BlockSpec((1, 1), lambda i: (0, 0)),
        ],
        out_specs=[pl.BlockSpec((RBS, HALF), lambda i: (i, 0)),
                   pl.BlockSpec((RBS, HALF), lambda i: (i, 0))],
        out_shape=[jax.ShapeDtypeStruct((N, HALF), _F32)] * 2,
    )(X, sa, sb, deg2, W, br, fl)


def _tc_pool_mlp(sa, sb, deg2, brow, bcol, Wm1, bm1r, Wm2, bm2r, C):
    def body(sa_ref, sb_ref, deg_ref, brow_ref, bcol_ref,
             wm1_ref, bm1_ref, wm2_ref, bm2_ref, out_ref,
             sum_s, max_s, cnt_s):
        i = pl.program_id(0)

        @pl.when(i == 0)
        def _():
            sum_s[...] = jnp.zeros_like(sum_s)
            cnt_s[...] = jnp.zeros_like(cnt_s)
            max_s[...] = jnp.full_like(max_s, -jnp.inf)

        dis = _dis_from_deg(deg_ref)
        x = jnp.concatenate([sa_ref[...], sb_ref[...]], axis=1) * dis
        x = jnp.maximum(x, 0.0)

        brw = brow_ref[0]
        oneh = (lax.broadcasted_iota(jnp.int32, (G, RB), 0) == brw
                ).astype(_F32)
        sum_s[...] += jnp.dot(oneh, x, preferred_element_type=_F32)
        cnt_s[...] += jnp.sum(oneh, axis=1, keepdims=True)

        bcl = bcol_ref[...]

        def gbody(g, _):
            m = jnp.where(bcl == g, x, -jnp.inf)
            mg = jnp.max(m, axis=0, keepdims=True)
            max_s[pl.ds(g, 1), :] = jnp.maximum(max_s[pl.ds(g, 1), :], mg)
            return 0

        lax.fori_loop(bcol_ref[0, 0], bcol_ref[RB - 1, 0] + 1, gbody, 0,
                      unroll=False)

        @pl.when(i == NBLK - 1)
        def _():
            mean = sum_s[...] / (cnt_s[...] + 1e-12)
            g64 = jnp.concatenate([mean, max_s[...]], axis=1)
            h = jnp.maximum(
                jnp.dot(g64, wm1_ref[...], preferred_element_type=_F32)
                + bm1_ref[...], 0.0)
            out_ref[...] = (jnp.dot(h, wm2_ref[...],
                                    preferred_element_type=_F32)
                            + bm2_ref[...])

    return pl.pallas_call(
        body,
        grid=(NBLK,),
        in_specs=[
            pl.BlockSpec((RB, HALF), lambda i: (i, 0)),
            pl.BlockSpec((RB, HALF), lambda i: (i, 0)),
            pl.BlockSpec((NSC, RB, DW), lambda i: (0, i, 0)),
            pl.BlockSpec((1, 1, RB), lambda i: (i, 0, 0)),
            pl.BlockSpec((RB, 1), lambda i: (i, 0)),
            pl.BlockSpec((2 * D, D), lambda i: (0, 0)),
            pl.BlockSpec((1, D), lambda i: (0, 0)),
            pl.BlockSpec((D, C), lambda i: (0, 0)),
            pl.BlockSpec((1, C), lambda i: (0, 0)),
        ],
        out_specs=pl.BlockSpec((G, C), lambda i: (0, 0)),
        out_shape=jax.ShapeDtypeStruct((G, C), _F32),
        scratch_shapes=[pltpu.VMEM((G, D), _F32),
                        pltpu.VMEM((G, D), _F32),
                        pltpu.VMEM((G, 1), _F32)],
    )(sa, sb, deg2, brow, bcol, Wm1, bm1r, Wm2, bm2r)


def kernel(X, EI, batch, num_graphs,
           W1, b1, W2, b2, W3, b3, Wm1, bm1, Wm2, bm2):
    C = Wm2.shape[1]
    row = EI[0]
    col = EI[1]
    rowr = row.reshape(ERWS, KC)
    colr = col.reshape(ERWS, KC)
    z8 = jnp.zeros((N, DW), _F32)
    z64 = jnp.zeros((N, HALF), _F32)
    ones125 = jnp.ones((KC, DW), _F32)

    deg2 = _sc_degree(rowr, z8, ones125)

    Wstack = jnp.stack([W1, W2, W3])
    bstack = jnp.stack([b1.reshape(1, -1), b2.reshape(1, -1),
                        b3.reshape(1, -1)])
    fstack = jnp.asarray([1.0, 0.0, 0.0], _F32).reshape(3, 1, 1)

    def step(carry, inp):
        sa, sb = carry
        W, br, fl = inp
        h2a, h2b = _tc_stage(X, sa, sb, deg2, W, br, fl)
        return _sc_prop(h2a, h2b, colr, rowr), None

    (sa, sb), _ = lax.scan(step, (z64, z64), (Wstack, bstack, fstack),
                           unroll=3)

    return _tc_pool_mlp(sa, sb, deg2, batch.reshape(NBLK, 1, RB),
                        batch.reshape(N, 1), Wm1, bm1.reshape(1, -1),
                        Wm2, bm2.reshape(1, -1), C)


# speedup vs baseline: 26.5261x; 1.0057x over previous
import functools

import jax
import jax.numpy as jnp
from jax import lax
from jax.experimental import pallas as pl
from jax.experimental.pallas import tpu as pltpu
from jax.experimental.pallas import tpu_sc as plsc

N = 10000
E = 320000
D = 128
HALF = 64
G = 64
KC = 125
ERWS = E // KC
NTEC = 16
NSC = 2
DW = 8
RB = 1000
NBLK = N // RB
SLAB = 624
SLAB_LAST = N - SLAB * (NTEC - 1)

_F32 = jnp.float32
_SC_PARAMS = pltpu.CompilerParams(use_tc_tiling_on_sc=False)


def _mesh():
    return plsc.VectorSubcoreMesh(core_axis_name="c", subcore_axis_name="s")


def _per_tec_slab(s, emit):
    base = pl.multiple_of(s * SLAB, 8)

    @pl.when(s < NTEC - 1)
    def _():
        emit(base, SLAB)

    @pl.when(s == NTEC - 1)
    def _():
        emit(SLAB * (NTEC - 1), SLAB_LAST)


def _sc_degree(rowr, z8, ones125):
    rpt = ERWS // (NSC * NTEC)

    @functools.partial(
        pl.kernel,
        mesh=_mesh(),
        out_type=jax.ShapeDtypeStruct((NSC, N, DW), _F32),
        compiler_params=_SC_PARAMS,
        scratch_types=[
            pltpu.VMEM((rpt, KC), jnp.int32),
            pltpu.VMEM((KC, DW), _F32),
            pltpu.VMEM_SHARED((N, DW), _F32),
            pltpu.SemaphoreType.DMA,
        ],
    )
    def k(rowr_h, z_h, ones_h, out_h, idx_v, ones_v, acc, sem):
        c = lax.axis_index("c")
        s = lax.axis_index("s")
        wid = c * NTEC + s
        pltpu.sync_copy(rowr_h.at[pl.ds(pl.multiple_of(wid * rpt, 8), rpt)],
                        idx_v)
        pltpu.sync_copy(ones_h, ones_v)
        _per_tec_slab(s, lambda o, n: pltpu.sync_copy(
            z_h.at[pl.ds(o, n)], acc.at[pl.ds(o, n)]))
        plsc.subcore_barrier()

        @pl.loop(0, rpt)
        def _(j):
            pltpu.sync_copy(ones_v, acc.at[idx_v.at[j]], add=True)

        plsc.subcore_barrier()
        _per_tec_slab(s, lambda o, n: pltpu.sync_copy(
            acc.at[pl.ds(o, n)], out_h.at[c, pl.ds(o, n)]))

    return k(rowr, z8, ones125)


def _sc_prop(h2a, h2b, colr, rowr):
    rpt = ERWS // NTEC

    @functools.partial(
        pl.kernel,
        mesh=_mesh(),
        out_type=(jax.ShapeDtypeStruct((N, HALF), _F32),
                  jax.ShapeDtypeStruct((N, HALF), _F32)),
        compiler_params=_SC_PARAMS,
        scratch_types=[
            pltpu.VMEM((rpt, KC), jnp.int32),
            pltpu.VMEM((rpt, KC), jnp.int32),
            [pltpu.VMEM((KC, HALF), _F32)] * 4,
            [pltpu.SemaphoreType.DMA] * 4,
            [pltpu.SemaphoreType.DMA] * 4,
            pltpu.VMEM_SHARED((N, HALF), _F32),
        ],
    )
    def k(h2a_h, h2b_h, colr_h, rowr_h, oa_h, ob_h,
          col_v, row_v, bufs, gsems, ssems, acc):
        c = lax.axis_index("c")
        s = lax.axis_index("s")
        e0 = pl.multiple_of(s * rpt, 8)
        pltpu.sync_copy(colr_h.at[pl.ds(e0, rpt)], col_v)
        pltpu.sync_copy(rowr_h.at[pl.ds(e0, rpt)], row_v)
        for cid in range(NSC):
            @pl.when(c == cid)
            def _(cid=cid):
                h2 = (h2a_h, h2b_h)[cid]
                out = (oa_h, ob_h)[cid]
                _per_tec_slab(s, lambda o, n: pltpu.sync_copy(
                    h2.at[pl.ds(o, n)], acc.at[pl.ds(o, n)]))
                plsc.subcore_barrier()

                pltpu.async_copy(h2.at[col_v.at[0]], bufs[0], gsems[0])
                pltpu.async_copy(h2.at[col_v.at[1]], bufs[1], gsems[1])

                @pl.loop(0, rpt, step=4)
                def _(base):
                    for b in range(4):
                        j = base + b
                        g = j + 2
                        bg = (b + 2) % 4

                        @pl.when(g < rpt)
                        def _():
                            @pl.when(g >= 4)
                            def _():
                                pltpu.make_async_copy(
                                    bufs[bg], acc.at[row_v.at[g - 4]],
                                    ssems[bg]).wait()

                            pltpu.async_copy(h2.at[col_v.at[g]],
                                             bufs[bg], gsems[bg])

                        pltpu.make_async_copy(h2.at[col_v.at[j]],
                                              bufs[b], gsems[b]).wait()
                        pltpu.async_copy(bufs[b], acc.at[row_v.at[j]],
                                         ssems[b], add=True)

                for b in range(4):
                    pltpu.make_async_copy(
                        bufs[b], acc.at[row_v.at[rpt - 4 + b]],
                        ssems[b]).wait()

                plsc.subcore_barrier()
                _per_tec_slab(s, lambda o, n: pltpu.sync_copy(
                    acc.at[pl.ds(o, n)], out.at[pl.ds(o, n)]))

    return k(h2a, h2b, colr, rowr)


def _dis_from_deg(deg_ref):
    deg = deg_ref[0, :, :1] + deg_ref[1, :, :1] + (1.0 + 1e-12)
    return lax.rsqrt(deg)


def _tc_stage1(X, deg2, W, br):
    RBS = 2000
    NBS = N // RBS

    def body(x_ref, deg_ref, w_ref, b_ref, oa_ref, ob_ref):
        dis = _dis_from_deg(deg_ref)
        h = jnp.dot(x_ref[...], w_ref[...],
                    preferred_element_type=_F32) + b_ref[...]
        h2 = h * dis
        oa_ref[...] = h2[:, :HALF]
        ob_ref[...] = h2[:, HALF:]

    return pl.pallas_call(
        body,
        grid=(NBS,),
        in_specs=[
            pl.BlockSpec((RBS, D), lambda i: (i, 0)),
            pl.BlockSpec((NSC, RBS, DW), lambda i: (0, i, 0)),
            pl.BlockSpec((D, D), lambda i: (0, 0)),
            pl.BlockSpec((1, D), lambda i: (0, 0)),
        ],
        out_specs=[pl.BlockSpec((RBS, HALF), lambda i: (i, 0)),
                   pl.BlockSpec((RBS, HALF), lambda i: (i, 0))],
        out_shape=[jax.ShapeDtypeStruct((N, HALF), _F32)] * 2,
    )(X, deg2, W, br)


def _tc_stage23(sa, sb, deg2, W, br):
    RBS = 2000
    NBS = N // RBS

    def body(sa_ref, sb_ref, deg_ref, w_ref, b_ref, oa_ref, ob_ref):
        dis = _dis_from_deg(deg_ref)
        x = jnp.concatenate([sa_ref[...], sb_ref[...]], axis=1) * dis
        x = jnp.maximum(x, 0.0)
        h = jnp.dot(x, w_ref[...], preferred_element_type=_F32) + b_ref[...]
        h2 = h * dis
        oa_ref[...] = h2[:, :HALF]
        ob_ref[...] = h2[:, HALF:]

    return pl.pallas_call(
        body,
        grid=(NBS,),
        in_specs=[
            pl.BlockSpec((RBS, HALF), lambda i: (i, 0)),
            pl.BlockSpec((RBS, HALF), lambda i: (i, 0)),
            pl.BlockSpec((NSC, RBS, DW), lambda i: (0, i, 0)),
            pl.BlockSpec((D, D), lambda i: (0, 0)),
            pl.BlockSpec((1, D), lambda i: (0, 0)),
        ],
        out_specs=[pl.BlockSpec((RBS, HALF), lambda i: (i, 0)),
                   pl.BlockSpec((RBS, HALF), lambda i: (i, 0))],
        out_shape=[jax.ShapeDtypeStruct((N, HALF), _F32)] * 2,
    )(sa, sb, deg2, W, br)


RBP = 400
NBP = N // RBP


def _tc_pool_mlp(sa, sb, deg2, brow, bcol, Wm1, bm1r, Wm2, bm2r, C):
    def body(sa_ref, sb_ref, deg_ref, brow_ref, bcol_ref,
             wm1_ref, bm1_ref, wm2_ref, bm2_ref, out_ref,
             sum_s, max_s, cnt_s):
        i = pl.program_id(0)

        @pl.when(i == 0)
        def _():
            sum_s[...] = jnp.zeros_like(sum_s)
            cnt_s[...] = jnp.zeros_like(cnt_s)
            max_s[...] = jnp.full_like(max_s, -jnp.inf)

        dis = _dis_from_deg(deg_ref)
        x = jnp.concatenate([sa_ref[...], sb_ref[...]], axis=1) * dis
        x = jnp.maximum(x, 0.0)

        brw = brow_ref[0]
        oneh = (lax.broadcasted_iota(jnp.int32, (G, RBP), 0) == brw
                ).astype(_F32)
        sum_s[...] += jnp.dot(oneh, x, preferred_element_type=_F32)
        cnt_s[...] += jnp.sum(oneh, axis=1, keepdims=True)

        bcl = bcol_ref[...]

        def gbody(g, _):
            m = jnp.where(bcl == g, x, -jnp.inf)
            mg = jnp.max(m, axis=0, keepdims=True)
            max_s[pl.ds(g, 1), :] = jnp.maximum(max_s[pl.ds(g, 1), :], mg)
            return 0

        lax.fori_loop(bcol_ref[0, 0], bcol_ref[RBP - 1, 0] + 1, gbody, 0,
                      unroll=False)

        @pl.when(i == NBP - 1)
        def _():
            mean = sum_s[...] / (cnt_s[...] + 1e-12)
            g64 = jnp.concatenate([mean, max_s[...]], axis=1)
            h = jnp.maximum(
                jnp.dot(g64, wm1_ref[...], preferred_element_type=_F32)
                + bm1_ref[...], 0.0)
            out_ref[...] = (jnp.dot(h, wm2_ref[...],
                                    preferred_element_type=_F32)
                            + bm2_ref[...])

    return pl.pallas_call(
        body,
        grid=(NBP,),
        in_specs=[
            pl.BlockSpec((RBP, HALF), lambda i: (i, 0)),
            pl.BlockSpec((RBP, HALF), lambda i: (i, 0)),
            pl.BlockSpec((NSC, RBP, DW), lambda i: (0, i, 0)),
            pl.BlockSpec((1, 1, RBP), lambda i: (i, 0, 0)),
            pl.BlockSpec((RBP, 1), lambda i: (i, 0)),
            pl.BlockSpec((2 * D, D), lambda i: (0, 0)),
            pl.BlockSpec((1, D), lambda i: (0, 0)),
            pl.BlockSpec((D, C), lambda i: (0, 0)),
            pl.BlockSpec((1, C), lambda i: (0, 0)),
        ],
        out_specs=pl.BlockSpec((G, C), lambda i: (0, 0)),
        out_shape=jax.ShapeDtypeStruct((G, C), _F32),
        scratch_shapes=[pltpu.VMEM((G, D), _F32),
                        pltpu.VMEM((G, D), _F32),
                        pltpu.VMEM((G, 1), _F32)],
    )(sa, sb, deg2, brow, bcol, Wm1, bm1r, Wm2, bm2r)


def kernel(X, EI, batch, num_graphs,
           W1, b1, W2, b2, W3, b3, Wm1, bm1, Wm2, bm2):
    C = Wm2.shape[1]
    row = EI[0]
    col = EI[1]
    rowr = row.reshape(ERWS, KC)
    colr = col.reshape(ERWS, KC)
    z8 = jnp.zeros((N, DW), _F32)
    ones125 = jnp.ones((KC, DW), _F32)

    deg2 = _sc_degree(rowr, z8, ones125)

    h2a, h2b = _tc_stage1(X, deg2, W1, b1.reshape(1, -1))
    sa, sb = _sc_prop(h2a, h2b, colr, rowr)
    h2a, h2b = _tc_stage23(sa, sb, deg2, W2, b2.reshape(1, -1))
    sa, sb = _sc_prop(h2a, h2b, colr, rowr)
    h2a, h2b = _tc_stage23(sa, sb, deg2, W3, b3.reshape(1, -1))
    sa, sb = _sc_prop(h2a, h2b, colr, rowr)

    return _tc_pool_mlp(sa, sb, deg2, batch.reshape(NBP, 1, RBP),
                        batch.reshape(N, 1), Wm1, bm1.reshape(1, -1),
                        Wm2, bm2.reshape(1, -1), C)


# speedup vs baseline: 27.2838x; 1.0286x over previous
import functools

import jax
import jax.numpy as jnp
from jax import lax
from jax.experimental import pallas as pl
from jax.experimental.pallas import tpu as pltpu
from jax.experimental.pallas import tpu_sc as plsc

N = 10000
E = 320000
D = 128
HALF = 64
G = 64
KC = 125
ERWS = E // KC
NTEC = 16
NSC = 2
DW = 8
RB = 1000
NBLK = N // RB
SLAB = 624
SLAB_LAST = N - SLAB * (NTEC - 1)

_F32 = jnp.float32
_SC_PARAMS = pltpu.CompilerParams(use_tc_tiling_on_sc=False)


def _mesh():
    return plsc.VectorSubcoreMesh(core_axis_name="c", subcore_axis_name="s")


def _per_tec_slab(s, emit):
    base = pl.multiple_of(s * SLAB, 8)

    @pl.when(s < NTEC - 1)
    def _():
        emit(base, SLAB)

    @pl.when(s == NTEC - 1)
    def _():
        emit(SLAB * (NTEC - 1), SLAB_LAST)


def _sc_degree(rowr, z8, ones125):
    rpt = ERWS // (NSC * NTEC)

    @functools.partial(
        pl.kernel,
        mesh=_mesh(),
        out_type=jax.ShapeDtypeStruct((NSC, N, DW), _F32),
        compiler_params=_SC_PARAMS,
        scratch_types=[
            pltpu.VMEM((rpt, KC), jnp.int32),
            pltpu.VMEM((KC, DW), _F32),
            pltpu.VMEM_SHARED((N, DW), _F32),
            pltpu.SemaphoreType.DMA,
        ],
    )
    def k(rowr_h, z_h, ones_h, out_h, idx_v, ones_v, acc, sem):
        c = lax.axis_index("c")
        s = lax.axis_index("s")
        wid = c * NTEC + s
        pltpu.sync_copy(rowr_h.at[pl.ds(pl.multiple_of(wid * rpt, 8), rpt)],
                        idx_v)
        pltpu.sync_copy(ones_h, ones_v)
        _per_tec_slab(s, lambda o, n: pltpu.sync_copy(
            z_h.at[pl.ds(o, n)], acc.at[pl.ds(o, n)]))
        plsc.subcore_barrier()

        @pl.loop(0, rpt)
        def _(j):
            pltpu.async_copy(ones_v, acc.at[idx_v.at[j]], sem, add=True)

        @pl.loop(0, rpt)
        def _(j):
            pltpu.make_async_copy(ones_v, acc.at[idx_v.at[j]], sem).wait()

        plsc.subcore_barrier()
        _per_tec_slab(s, lambda o, n: pltpu.sync_copy(
            acc.at[pl.ds(o, n)], out_h.at[c, pl.ds(o, n)]))

    return k(rowr, z8, ones125)


def _sc_prop(h2a, h2b, colr, rowr):
    rpt = ERWS // NTEC

    @functools.partial(
        pl.kernel,
        mesh=_mesh(),
        out_type=(jax.ShapeDtypeStruct((N, HALF), _F32),
                  jax.ShapeDtypeStruct((N, HALF), _F32)),
        compiler_params=_SC_PARAMS,
        scratch_types=[
            pltpu.VMEM((rpt, KC), jnp.int32),
            pltpu.VMEM((rpt, KC), jnp.int32),
            [pltpu.VMEM((KC, HALF), _F32)] * 4,
            [pltpu.SemaphoreType.DMA] * 4,
            [pltpu.SemaphoreType.DMA] * 4,
            pltpu.VMEM_SHARED((N, HALF), _F32),
        ],
    )
    def k(h2a_h, h2b_h, colr_h, rowr_h, oa_h, ob_h,
          col_v, row_v, bufs, gsems, ssems, acc):
        c = lax.axis_index("c")
        s = lax.axis_index("s")
        e0 = pl.multiple_of(s * rpt, 8)
        pltpu.async_copy(colr_h.at[pl.ds(e0, rpt)], col_v, gsems[2])
        pltpu.async_copy(rowr_h.at[pl.ds(e0, rpt)], row_v, gsems[3])
        for cid in range(NSC):
            @pl.when(c == cid)
            def _(cid=cid):
                h2 = (h2a_h, h2b_h)[cid]
                out = (oa_h, ob_h)[cid]
                _per_tec_slab(s, lambda o, n: pltpu.async_copy(
                    h2.at[pl.ds(o, n)], acc.at[pl.ds(o, n)], ssems[0]))
                pltpu.make_async_copy(colr_h.at[pl.ds(e0, rpt)], col_v,
                                      gsems[2]).wait()
                pltpu.async_copy(h2.at[col_v.at[0]], bufs[0], gsems[0])
                pltpu.async_copy(h2.at[col_v.at[1]], bufs[1], gsems[1])
                pltpu.make_async_copy(rowr_h.at[pl.ds(e0, rpt)], row_v,
                                      gsems[3]).wait()
                _per_tec_slab(s, lambda o, n: pltpu.make_async_copy(
                    h2.at[pl.ds(o, n)], acc.at[pl.ds(o, n)],
                    ssems[0]).wait())
                plsc.subcore_barrier()

                @pl.loop(0, rpt, step=4)
                def _(base):
                    for b in range(4):
                        j = base + b
                        g = j + 2
                        bg = (b + 2) % 4

                        @pl.when(g < rpt)
                        def _():
                            @pl.when(g >= 4)
                            def _():
                                pltpu.make_async_copy(
                                    bufs[bg], acc.at[row_v.at[g - 4]],
                                    ssems[bg]).wait()

                            pltpu.async_copy(h2.at[col_v.at[g]],
                                             bufs[bg], gsems[bg])

                        pltpu.make_async_copy(h2.at[col_v.at[j]],
                                              bufs[b], gsems[b]).wait()
                        pltpu.async_copy(bufs[b], acc.at[row_v.at[j]],
                                         ssems[b], add=True)

                for b in range(4):
                    pltpu.make_async_copy(
                        bufs[b], acc.at[row_v.at[rpt - 4 + b]],
                        ssems[b]).wait()

                plsc.subcore_barrier()
                _per_tec_slab(s, lambda o, n: pltpu.sync_copy(
                    acc.at[pl.ds(o, n)], out.at[pl.ds(o, n)]))

    return k(h2a, h2b, colr, rowr)


def _dis_from_deg(deg_ref):
    deg = deg_ref[0, :, :1] + deg_ref[1, :, :1] + (1.0 + 1e-12)
    return lax.rsqrt(deg)


def _tc_stage1(X, deg2, W, br):
    RBS = 2000
    NBS = N // RBS

    def body(x_ref, deg_ref, w_ref, b_ref, oa_ref, ob_ref):
        dis = _dis_from_deg(deg_ref)
        h = jnp.dot(x_ref[...], w_ref[...],
                    preferred_element_type=_F32) + b_ref[...]
        h2 = h * dis
        oa_ref[...] = h2[:, :HALF]
        ob_ref[...] = h2[:, HALF:]

    return pl.pallas_call(
        body,
        grid=(NBS,),
        in_specs=[
            pl.BlockSpec((RBS, D), lambda i: (i, 0)),
            pl.BlockSpec((NSC, RBS, DW), lambda i: (0, i, 0)),
            pl.BlockSpec((D, D), lambda i: (0, 0)),
            pl.BlockSpec((1, D), lambda i: (0, 0)),
        ],
        out_specs=[pl.BlockSpec((RBS, HALF), lambda i: (i, 0)),
                   pl.BlockSpec((RBS, HALF), lambda i: (i, 0))],
        out_shape=[jax.ShapeDtypeStruct((N, HALF), _F32)] * 2,
    )(X, deg2, W, br)


def _tc_stage23(sa, sb, deg2, W, br):
    RBS = 2000
    NBS = N // RBS

    def body(sa_ref, sb_ref, deg_ref, w_ref, b_ref, oa_ref, ob_ref):
        dis = _dis_from_deg(deg_ref)
        x = jnp.concatenate([sa_ref[...], sb_ref[...]], axis=1) * dis
        x = jnp.maximum(x, 0.0)
        h = jnp.dot(x, w_ref[...], preferred_element_type=_F32) + b_ref[...]
        h2 = h * dis
        oa_ref[...] = h2[:, :HALF]
        ob_ref[...] = h2[:, HALF:]

    return pl.pallas_call(
        body,
        grid=(NBS,),
        in_specs=[
            pl.BlockSpec((RBS, HALF), lambda i: (i, 0)),
            pl.BlockSpec((RBS, HALF), lambda i: (i, 0)),
            pl.BlockSpec((NSC, RBS, DW), lambda i: (0, i, 0)),
            pl.BlockSpec((D, D), lambda i: (0, 0)),
            pl.BlockSpec((1, D), lambda i: (0, 0)),
        ],
        out_specs=[pl.BlockSpec((RBS, HALF), lambda i: (i, 0)),
                   pl.BlockSpec((RBS, HALF), lambda i: (i, 0))],
        out_shape=[jax.ShapeDtypeStruct((N, HALF), _F32)] * 2,
    )(sa, sb, deg2, W, br)


RBP = 400
NBP = N // RBP


def _tc_pool_mlp(sa, sb, deg2, brow, bcol, Wm1, bm1r, Wm2, bm2r, C):
    def body(sa_ref, sb_ref, deg_ref, brow_ref, bcol_ref,
             wm1_ref, bm1_ref, wm2_ref, bm2_ref, out_ref,
             sum_s, max_s, cnt_s):
        i = pl.program_id(0)

        @pl.when(i == 0)
        def _():
            sum_s[...] = jnp.zeros_like(sum_s)
            cnt_s[...] = jnp.zeros_like(cnt_s)
            max_s[...] = jnp.full_like(max_s, -jnp.inf)

        dis = _dis_from_deg(deg_ref)
        x = jnp.concatenate([sa_ref[...], sb_ref[...]], axis=1) * dis
        x = jnp.maximum(x, 0.0)

        brw = brow_ref[0]
        oneh = (lax.broadcasted_iota(jnp.int32, (G, RBP), 0) == brw
                ).astype(_F32)
        sum_s[...] += jnp.dot(oneh, x, preferred_element_type=_F32)
        cnt_s[...] += jnp.sum(oneh, axis=1, keepdims=True)

        bcl = bcol_ref[...]

        def gbody(g, _):
            m = jnp.where(bcl == g, x, -jnp.inf)
            mg = jnp.max(m, axis=0, keepdims=True)
            max_s[pl.ds(g, 1), :] = jnp.maximum(max_s[pl.ds(g, 1), :], mg)
            return 0

        lax.fori_loop(bcol_ref[0, 0], bcol_ref[RBP - 1, 0] + 1, gbody, 0,
                      unroll=False)

        @pl.when(i == NBP - 1)
        def _():
            mean = sum_s[...] / (cnt_s[...] + 1e-12)
            g64 = jnp.concatenate([mean, max_s[...]], axis=1)
            h = jnp.maximum(
                jnp.dot(g64, wm1_ref[...], preferred_element_type=_F32)
                + bm1_ref[...], 0.0)
            out_ref[...] = (jnp.dot(h, wm2_ref[...],
                                    preferred_element_type=_F32)
                            + bm2_ref[...])

    return pl.pallas_call(
        body,
        grid=(NBP,),
        in_specs=[
            pl.BlockSpec((RBP, HALF), lambda i: (i, 0)),
            pl.BlockSpec((RBP, HALF), lambda i: (i, 0)),
            pl.BlockSpec((NSC, RBP, DW), lambda i: (0, i, 0)),
            pl.BlockSpec((1, 1, RBP), lambda i: (i, 0, 0)),
            pl.BlockSpec((RBP, 1), lambda i: (i, 0)),
            pl.BlockSpec((2 * D, D), lambda i: (0, 0)),
            pl.BlockSpec((1, D), lambda i: (0, 0)),
            pl.BlockSpec((D, C), lambda i: (0, 0)),
            pl.BlockSpec((1, C), lambda i: (0, 0)),
        ],
        out_specs=pl.BlockSpec((G, C), lambda i: (0, 0)),
        out_shape=jax.ShapeDtypeStruct((G, C), _F32),
        scratch_shapes=[pltpu.VMEM((G, D), _F32),
                        pltpu.VMEM((G, D), _F32),
                        pltpu.VMEM((G, 1), _F32)],
    )(sa, sb, deg2, brow, bcol, Wm1, bm1r, Wm2, bm2r)


def kernel(X, EI, batch, num_graphs,
           W1, b1, W2, b2, W3, b3, Wm1, bm1, Wm2, bm2):
    C = Wm2.shape[1]
    row = EI[0]
    col = EI[1]
    rowr = row.reshape(ERWS, KC)
    colr = col.reshape(ERWS, KC)
    z8 = jnp.zeros((N, DW), _F32)
    ones125 = jnp.ones((KC, DW), _F32)

    deg2 = _sc_degree(rowr, z8, ones125)

    h2a, h2b = _tc_stage1(X, deg2, W1, b1.reshape(1, -1))
    sa, sb = _sc_prop(h2a, h2b, colr, rowr)
    h2a, h2b = _tc_stage23(sa, sb, deg2, W2, b2.reshape(1, -1))
    sa, sb = _sc_prop(h2a, h2b, colr, rowr)
    h2a, h2b = _tc_stage23(sa, sb, deg2, W3, b3.reshape(1, -1))
    sa, sb = _sc_prop(h2a, h2b, colr, rowr)

    return _tc_pool_mlp(sa, sb, deg2, batch.reshape(NBP, 1, RBP),
                        batch.reshape(N, 1), Wm1, bm1.reshape(1, -1),
                        Wm2, bm2.reshape(1, -1), C)


# speedup vs baseline: 27.3016x; 1.0006x over previous
import functools

import jax
import jax.numpy as jnp
from jax import lax
from jax.experimental import pallas as pl
from jax.experimental.pallas import tpu as pltpu
from jax.experimental.pallas import tpu_sc as plsc

N = 10000
E = 320000
D = 128
HALF = 64
G = 64
KC = 125
ERWS = E // KC
NTEC = 16
NSC = 2
DW = 8
SLAB = 624
SLAB_LAST = N - SLAB * (NTEC - 1)

_F32 = jnp.float32
_SC_PARAMS = pltpu.CompilerParams(use_tc_tiling_on_sc=False)


def _mesh():
    return plsc.VectorSubcoreMesh(core_axis_name="c", subcore_axis_name="s")


def _per_tec_slab(s, emit):
    base = pl.multiple_of(s * SLAB, 8)

    @pl.when(s < NTEC - 1)
    def _():
        emit(base, SLAB)

    @pl.when(s == NTEC - 1)
    def _():
        emit(SLAB * (NTEC - 1), SLAB_LAST)


def _sc_degree(rowr, z8, ones125):
    rpt = ERWS // (NSC * NTEC)

    @functools.partial(
        pl.kernel,
        mesh=_mesh(),
        out_type=jax.ShapeDtypeStruct((NSC, N, DW), _F32),
        compiler_params=_SC_PARAMS,
        scratch_types=[
            pltpu.VMEM((rpt, KC), jnp.int32),
            pltpu.VMEM((KC, DW), _F32),
            pltpu.VMEM_SHARED((N, DW), _F32),
            pltpu.SemaphoreType.DMA,
        ],
    )
    def k(rowr_h, z_h, ones_h, out_h, idx_v, ones_v, acc, sem):
        c = lax.axis_index("c")
        s = lax.axis_index("s")
        wid = c * NTEC + s
        pltpu.sync_copy(rowr_h.at[pl.ds(pl.multiple_of(wid * rpt, 8), rpt)],
                        idx_v)
        pltpu.sync_copy(ones_h, ones_v)
        _per_tec_slab(s, lambda o, n: pltpu.sync_copy(
            z_h.at[pl.ds(o, n)], acc.at[pl.ds(o, n)]))
        plsc.subcore_barrier()

        @pl.loop(0, rpt)
        def _(j):
            pltpu.async_copy(ones_v, acc.at[idx_v.at[j]], sem, add=True)

        @pl.loop(0, rpt)
        def _(j):
            pltpu.make_async_copy(ones_v, acc.at[idx_v.at[j]], sem).wait()

        plsc.subcore_barrier()
        _per_tec_slab(s, lambda o, n: pltpu.sync_copy(
            acc.at[pl.ds(o, n)], out_h.at[c, pl.ds(o, n)]))

    return k(rowr, z8, ones125)


def _sc_prop(h2a, h2b, colr, rowr):
    rpt = ERWS // NTEC

    @functools.partial(
        pl.kernel,
        mesh=_mesh(),
        out_type=(jax.ShapeDtypeStruct((N, HALF), _F32),
                  jax.ShapeDtypeStruct((N, HALF), _F32)),
        compiler_params=_SC_PARAMS,
        scratch_types=[
            pltpu.VMEM((rpt, KC), jnp.int32),
            pltpu.VMEM((rpt, KC), jnp.int32),
            [pltpu.VMEM((KC, HALF), _F32)] * 4,
            [pltpu.SemaphoreType.DMA] * 4,
            [pltpu.SemaphoreType.DMA] * 4,
            pltpu.VMEM_SHARED((N, HALF), _F32),
        ],
    )
    def k(h2a_h, h2b_h, colr_h, rowr_h, oa_h, ob_h,
          col_v, row_v, bufs, gsems, ssems, acc):
        c = lax.axis_index("c")
        s = lax.axis_index("s")
        e0 = pl.multiple_of(s * rpt, 8)
        pltpu.async_copy(colr_h.at[pl.ds(e0, rpt)], col_v, gsems[2])
        pltpu.async_copy(rowr_h.at[pl.ds(e0, rpt)], row_v, gsems[3])
        for cid in range(NSC):
            @pl.when(c == cid)
            def _(cid=cid):
                h2 = (h2a_h, h2b_h)[cid]
                out = (oa_h, ob_h)[cid]
                _per_tec_slab(s, lambda o, n: pltpu.async_copy(
                    h2.at[pl.ds(o, n)], acc.at[pl.ds(o, n)], ssems[0]))
                pltpu.make_async_copy(colr_h.at[pl.ds(e0, rpt)], col_v,
                                      gsems[2]).wait()
                pltpu.async_copy(h2.at[col_v.at[0]], bufs[0], gsems[0])
                pltpu.async_copy(h2.at[col_v.at[1]], bufs[1], gsems[1])
                pltpu.make_async_copy(rowr_h.at[pl.ds(e0, rpt)], row_v,
                                      gsems[3]).wait()
                _per_tec_slab(s, lambda o, n: pltpu.make_async_copy(
                    h2.at[pl.ds(o, n)], acc.at[pl.ds(o, n)],
                    ssems[0]).wait())
                plsc.subcore_barrier()

                @pl.loop(0, rpt, step=4)
                def _(base):
                    for b in range(4):
                        j = base + b
                        g = j + 2
                        bg = (b + 2) % 4

                        @pl.when(g < rpt)
                        def _():
                            @pl.when(g >= 4)
                            def _():
                                pltpu.make_async_copy(
                                    bufs[bg], acc.at[row_v.at[g - 4]],
                                    ssems[bg]).wait()

                            pltpu.async_copy(h2.at[col_v.at[g]],
                                             bufs[bg], gsems[bg])

                        pltpu.make_async_copy(h2.at[col_v.at[j]],
                                              bufs[b], gsems[b]).wait()
                        pltpu.async_copy(bufs[b], acc.at[row_v.at[j]],
                                         ssems[b], add=True)

                for b in range(4):
                    pltpu.make_async_copy(
                        bufs[b], acc.at[row_v.at[rpt - 4 + b]],
                        ssems[b]).wait()

                plsc.subcore_barrier()
                _per_tec_slab(s, lambda o, n: pltpu.sync_copy(
                    acc.at[pl.ds(o, n)], out.at[pl.ds(o, n)]))

    return k(h2a, h2b, colr, rowr)


def _dis_from_deg(deg_ref):
    deg = deg_ref[0, :, :1] + deg_ref[1, :, :1] + (1.0 + 1e-12)
    return lax.rsqrt(deg)


def _tc_stage1(X, deg2, W, br):
    RBS = 2000
    NBS = N // RBS

    def body(x_ref, deg_ref, w_ref, b_ref, oa_ref, ob_ref):
        dis = _dis_from_deg(deg_ref)
        h = jnp.dot(x_ref[...], w_ref[...],
                    preferred_element_type=_F32) + b_ref[...]
        h2 = h * dis
        oa_ref[...] = h2[:, :HALF]
        ob_ref[...] = h2[:, HALF:]

    return pl.pallas_call(
        body,
        grid=(NBS,),
        in_specs=[
            pl.BlockSpec((RBS, D), lambda i: (i, 0)),
            pl.BlockSpec((NSC, RBS, DW), lambda i: (0, i, 0)),
            pl.BlockSpec((D, D), lambda i: (0, 0)),
            pl.BlockSpec((1, D), lambda i: (0, 0)),
        ],
        out_specs=[pl.BlockSpec((RBS, HALF), lambda i: (i, 0)),
                   pl.BlockSpec((RBS, HALF), lambda i: (i, 0))],
        out_shape=[jax.ShapeDtypeStruct((N, HALF), _F32)] * 2,
    )(X, deg2, W, br)


def _tc_stage23(sa, sb, deg2, W, br):
    RBS = 2000
    NBS = N // RBS

    def body(sa_ref, sb_ref, deg_ref, w_ref, b_ref, oa_ref, ob_ref):
        dis = _dis_from_deg(deg_ref)
        x = jnp.concatenate([sa_ref[...], sb_ref[...]], axis=1) * dis
        x = jnp.maximum(x, 0.0)
        h = jnp.dot(x, w_ref[...], preferred_element_type=_F32) + b_ref[...]
        h2 = h * dis
        oa_ref[...] = h2[:, :HALF]
        ob_ref[...] = h2[:, HALF:]

    return pl.pallas_call(
        body,
        grid=(NBS,),
        in_specs=[
            pl.BlockSpec((RBS, HALF), lambda i: (i, 0)),
            pl.BlockSpec((RBS, HALF), lambda i: (i, 0)),
            pl.BlockSpec((NSC, RBS, DW), lambda i: (0, i, 0)),
            pl.BlockSpec((D, D), lambda i: (0, 0)),
            pl.BlockSpec((1, D), lambda i: (0, 0)),
        ],
        out_specs=[pl.BlockSpec((RBS, HALF), lambda i: (i, 0)),
                   pl.BlockSpec((RBS, HALF), lambda i: (i, 0))],
        out_shape=[jax.ShapeDtypeStruct((N, HALF), _F32)] * 2,
    )(sa, sb, deg2, W, br)


RBP = 400
NBP = N // RBP


def _tc_pool_mlp(sa, sb, deg2, brow, bcol, Wm1, bm1r, Wm2, bm2r, C):
    def body(sa_ref, sb_ref, deg_ref, brow_ref, bcol_ref,
             wm1_ref, bm1_ref, wm2_ref, bm2_ref, out_ref,
             sum_s, max_s, cnt_s):
        i = pl.program_id(0)

        @pl.when(i == 0)
        def _():
            sum_s[...] = jnp.zeros_like(sum_s)
            cnt_s[...] = jnp.zeros_like(cnt_s)
            max_s[...] = jnp.full_like(max_s, -jnp.inf)

        dis = _dis_from_deg(deg_ref)
        x = jnp.concatenate([sa_ref[...], sb_ref[...]], axis=1) * dis
        x = jnp.maximum(x, 0.0)

        brw = brow_ref[0]
        oneh = (lax.broadcasted_iota(jnp.int32, (G, RBP), 0) == brw
                ).astype(_F32)
        sum_s[...] += jnp.dot(oneh, x, preferred_element_type=_F32)
        cnt_s[...] += jnp.sum(oneh, axis=1, keepdims=True)

        bcl = bcol_ref[...]

        def gbody(g, _):
            m = jnp.where(bcl == g, x, -jnp.inf)
            mg = jnp.max(m, axis=0, keepdims=True)
            max_s[pl.ds(g, 1), :] = jnp.maximum(max_s[pl.ds(g, 1), :], mg)
            return 0

        lax.fori_loop(bcol_ref[0, 0], bcol_ref[RBP - 1, 0] + 1, gbody, 0,
                      unroll=False)

        @pl.when(i == NBP - 1)
        def _():
            mean = sum_s[...] / (cnt_s[...] + 1e-12)
            g64 = jnp.concatenate([mean, max_s[...]], axis=1)
            h = jnp.maximum(
                jnp.dot(g64, wm1_ref[...], preferred_element_type=_F32)
                + bm1_ref[...], 0.0)
            out_ref[...] = (jnp.dot(h, wm2_ref[...],
                                    preferred_element_type=_F32)
                            + bm2_ref[...])

    return pl.pallas_call(
        body,
        grid=(NBP,),
        in_specs=[
            pl.BlockSpec((RBP, HALF), lambda i: (i, 0)),
            pl.BlockSpec((RBP, HALF), lambda i: (i, 0)),
            pl.BlockSpec((NSC, RBP, DW), lambda i: (0, i, 0)),
            pl.BlockSpec((1, 1, RBP), lambda i: (i, 0, 0)),
            pl.BlockSpec((RBP, 1), lambda i: (i, 0)),
            pl.BlockSpec((2 * D, D), lambda i: (0, 0)),
            pl.BlockSpec((1, D), lambda i: (0, 0)),
            pl.BlockSpec((D, C), lambda i: (0, 0)),
            pl.BlockSpec((1, C), lambda i: (0, 0)),
        ],
        out_specs=pl.BlockSpec((G, C), lambda i: (0, 0)),
        out_shape=jax.ShapeDtypeStruct((G, C), _F32),
        scratch_shapes=[pltpu.VMEM((G, D), _F32),
                        pltpu.VMEM((G, D), _F32),
                        pltpu.VMEM((G, 1), _F32)],
    )(sa, sb, deg2, brow, bcol, Wm1, bm1r, Wm2, bm2r)


def kernel(X, EI, batch, num_graphs,
           W1, b1, W2, b2, W3, b3, Wm1, bm1, Wm2, bm2):
    C = Wm2.shape[1]
    row = EI[0]
    col = EI[1]
    rowr = row.reshape(ERWS, KC)
    colr = col.reshape(ERWS, KC)
    z8 = jnp.zeros((N, DW), _F32)
    ones125 = jnp.ones((KC, DW), _F32)

    deg2 = _sc_degree(rowr, z8, ones125)

    h2a, h2b = _tc_stage1(X, deg2, W1, b1.reshape(1, -1))
    sa, sb = _sc_prop(h2a, h2b, colr, rowr)
    h2a, h2b = _tc_stage23(sa, sb, deg2, W2, b2.reshape(1, -1))
    sa, sb = _sc_prop(h2a, h2b, colr, rowr)
    h2a, h2b = _tc_stage23(sa, sb, deg2, W3, b3.reshape(1, -1))
    sa, sb = _sc_prop(h2a, h2b, colr, rowr)

    return _tc_pool_mlp(sa, sb, deg2, batch.reshape(NBP, 1, RBP),
                        batch.reshape(N, 1), Wm1, bm1.reshape(1, -1),
                        Wm2, bm2.reshape(1, -1), C)
